# baseline (speedup 1.0000x reference)
import functools

import jax
import jax.numpy as jnp
from jax import lax
from jax.experimental import pallas as pl
from jax.experimental.pallas import tpu as pltpu
from jax.experimental.pallas import tpu_sc as plsc


_BR = 256
_TL = 128


def _tc_body(w_ref, x_ref, o_ref, *, br, n, l):
    w = w_ref[...]
    carry = jnp.zeros((br, l), dtype=jnp.float32)
    for c in range(n // l):
        xc = x_ref[:, c * l : (c + 1) * l]
        y = jnp.dot(xc, w, preferred_element_type=jnp.float32)
        o_ref[:, c * l : (c + 1) * l] = y[:, :l] + carry
        carry = carry + y[:, l:]


def _tc_cumsum(x):
    m, n = x.shape
    tri = jnp.triu(jnp.ones((_TL, _TL), dtype=jnp.float32))
    w = jnp.concatenate([tri, jnp.ones((_TL, _TL), dtype=jnp.float32)], axis=1)
    return pl.pallas_call(
        functools.partial(_tc_body, br=_BR, n=n, l=_TL),
        grid=(m // _BR,),
        in_specs=[
            pl.BlockSpec((_TL, 2 * _TL), lambda i: (0, 0)),
            pl.BlockSpec((_BR, n), lambda i: (i, 0)),
        ],
        out_specs=pl.BlockSpec((_BR, n), lambda i: (i, 0)),
        out_shape=jax.ShapeDtypeStruct((m, n), x.dtype),
        compiler_params=pltpu.CompilerParams(
            dimension_semantics=("parallel",)
        ),
    )(w, x)


_INFO = plsc.get_sparse_core_info()
_NC = _INFO.num_cores
_NS = _INFO.num_subcores
_NW = _NC * _NS
_L = _INFO.num_lanes
_RB = 2


def _make_sc_kernel(m, n):
    rows_per_w = m // _NW
    nb = rows_per_w // _RB
    nb2 = nb // 2
    nchunks = n // _L
    mesh = plsc.VectorSubcoreMesh(core_axis_name="c", subcore_axis_name="s")

    @functools.partial(
        pl.kernel,
        mesh=mesh,
        out_type=jax.ShapeDtypeStruct((m, n), jnp.float32),
        scratch_types=[
            pltpu.VMEM((_RB, n), jnp.float32),
            pltpu.VMEM((_RB, n), jnp.float32),
            pltpu.VMEM((_RB, n), jnp.float32),
            pltpu.VMEM((_RB, n), jnp.float32),
            pltpu.SemaphoreType.DMA,
            pltpu.SemaphoreType.DMA,
            pltpu.SemaphoreType.DMA,
            pltpu.SemaphoreType.DMA,
        ],
        compiler_params=pltpu.CompilerParams(needs_layout_passes=False),
    )
    def k(x_hbm, o_hbm, in0, in1, out0, out1, si0, si1, so0, so1):
        wid = lax.axis_index("s") * _NC + lax.axis_index("c")
        base = wid * rows_per_w

        def rows(b):
            return base + b * _RB

        def compute(in_v, out_v):
            def chunk_body(c, carrys):
                last = jnp.full((_L,), _L - 1, dtype=jnp.int32)
                new_carrys = []
                for r in range(_RB):
                    chunk = in_v[r, pl.ds(c * _L, _L)]
                    s = plsc.cumsum(chunk) + carrys[r]
                    out_v[r, pl.ds(c * _L, _L)] = s
                    new_carrys.append(
                        jnp.take_along_axis(
                            s, last, axis=0, mode="promise_in_bounds"
                        )
                    )
                return tuple(new_carrys)

            lax.fori_loop(
                0, nchunks, chunk_body,
                tuple(jnp.zeros((_L,), jnp.float32) for _ in range(_RB)),
            )

        pltpu.async_copy(x_hbm.at[pl.ds(rows(0), _RB)], in0, si0)

        def body(p, _):
            b0 = 2 * p
            b1 = b0 + 1
            pltpu.async_copy(x_hbm.at[pl.ds(rows(b1), _RB)], in1, si1)
            pltpu.make_async_copy(
                x_hbm.at[pl.ds(rows(b0), _RB)], in0, si0
            ).wait()

            @pl.when(p > 0)
            def _():
                pltpu.make_async_copy(
                    out0, o_hbm.at[pl.ds(rows(b0 - 2), _RB)], so0
                ).wait()

            compute(in0, out0)
            pltpu.async_copy(out0, o_hbm.at[pl.ds(rows(b0), _RB)], so0)

            @pl.when(p < nb2 - 1)
            def _():
                pltpu.async_copy(
                    x_hbm.at[pl.ds(rows(b0 + 2), _RB)], in0, si0
                )

            pltpu.make_async_copy(
                x_hbm.at[pl.ds(rows(b1), _RB)], in1, si1
            ).wait()

            @pl.when(p > 0)
            def _():
                pltpu.make_async_copy(
                    out1, o_hbm.at[pl.ds(rows(b1 - 2), _RB)], so1
                ).wait()

            compute(in1, out1)
            pltpu.async_copy(out1, o_hbm.at[pl.ds(rows(b1), _RB)], so1)
            return 0

        lax.fori_loop(0, nb2, body, 0)
        pltpu.make_async_copy(
            out0, o_hbm.at[pl.ds(rows(nb - 2), _RB)], so0
        ).wait()
        pltpu.make_async_copy(
            out1, o_hbm.at[pl.ds(rows(nb - 1), _RB)], so1
        ).wait()

    return k


@jax.jit
def kernel(x):
    m, n = x.shape
    a = _tc_cumsum(x)
    b = _make_sc_kernel(m, n)(x)
    return a, b


# speedup vs baseline: 2.1771x; 2.1771x over previous
import functools

import jax
import jax.numpy as jnp
from jax.experimental import pallas as pl
from jax.experimental.pallas import tpu as pltpu

_BR = 256
_L = 128


def _cumsum_kernel(w_ref, x_ref, o_ref, *, br, n, l):
    w = w_ref[...]
    carry = jnp.zeros((br, l), dtype=jnp.float32)
    for c in range(n // l):
        xc = x_ref[:, c * l : (c + 1) * l]
        y = jnp.dot(xc, w, preferred_element_type=jnp.float32)
        o_ref[:, c * l : (c + 1) * l] = y[:, :l] + carry
        carry = carry + y[:, l:]


@jax.jit
def kernel(x):
    m, n = x.shape
    tri = jnp.triu(jnp.ones((_L, _L), dtype=jnp.float32))
    w = jnp.concatenate([tri, jnp.ones((_L, _L), dtype=jnp.float32)], axis=1)
    return pl.pallas_call(
        functools.partial(_cumsum_kernel, br=_BR, n=n, l=_L),
        grid=(m // _BR,),
        in_specs=[
            pl.BlockSpec((_L, 2 * _L), lambda i: (0, 0)),
            pl.BlockSpec((_BR, n), lambda i: (i, 0)),
        ],
        out_specs=pl.BlockSpec((_BR, n), lambda i: (i, 0)),
        out_shape=jax.ShapeDtypeStruct((m, n), x.dtype),
        compiler_params=pltpu.CompilerParams(
            dimension_semantics=("parallel",)
        ),
    )(w, x)
